# Initial kernel scaffold; baseline (speedup 1.0000x reference)
import jax
import jax.numpy as jnp
from jax.experimental import pallas as pl


def kernel(x, edge_index, W1, b1, W2, b2, W3, b3):
    raise NotImplementedError("write your pallas kernel here")


# speedup vs baseline: 24.4898x; 24.4898x over previous
import functools

import jax
import jax.numpy as jnp
from jax import lax
from jax.experimental import pallas as pl
from jax.experimental.pallas import tpu as pltpu
from jax.experimental.pallas import tpu_sc as plsc

N_NODES = 100000
N_PAD = 102400
DUMMY = N_NODES
LANES = 16
E_CHUNK = 128
KD = 8
E_ROWS = 12544
ZROWS = 1600
R_TC = 2048
NEG = -1e30


def _make_scatter_kernel(mode):
    col_split = mode == "agg32"
    n_workers = 16 if col_split else 32
    rows_per_tile = E_ROWS // n_workers
    groups = rows_per_tile // KD
    mesh = plsc.VectorSubcoreMesh(core_axis_name="c", subcore_axis_name="s")

    def body(table, srcr, dstr, e0c, out, src_v, dst_v, dbuf, accum, sem):
        cid = lax.axis_index("c")
        sid = lax.axis_index("s")
        tix = cid if col_split else 0
        for t in range(4):
            pltpu.sync_copy(table.at[tix, pl.ds(N_NODES, ZROWS)],
                            accum.at[pl.ds(sid * 6400 + t * ZROWS, ZROWS)])
        if mode == "deg":
            pltpu.sync_copy(e0c, dbuf.at[0])
        plsc.subcore_barrier()

        if col_split:
            row0 = sid * rows_per_tile
        else:
            row0 = (cid * 16 + sid) * rows_per_tile

        def grp(g, carry):
            base = row0 + g * KD
            pltpu.sync_copy(dstr.at[pl.ds(base, KD)], dst_v)
            if mode == "deg":
                for j in range(KD):
                    pltpu.sync_copy(dbuf.at[0], accum.at[dst_v.at[j]], add=True)
            else:
                pltpu.sync_copy(srcr.at[pl.ds(base, KD)], src_v)
                descs = [
                    pltpu.async_copy(table.at[tix].at[src_v.at[j]], dbuf.at[j], sem)
                    for j in range(KD)
                ]
                for dd in descs:
                    dd.wait()
                for j in range(KD):
                    pltpu.sync_copy(dbuf.at[j], accum.at[dst_v.at[j]], add=True)
            return carry

        lax.fori_loop(0, groups, grp, 0)
        plsc.subcore_barrier()
        for t in range(4):
            sl = pl.ds(sid * 6400 + t * ZROWS, ZROWS)
            pltpu.sync_copy(accum.at[sl], out.at[cid, sl])

    return pl.kernel(
        body,
        out_type=jax.ShapeDtypeStruct((2, N_PAD, LANES), jnp.float32),
        mesh=mesh,
        scratch_types=[
            pltpu.VMEM((KD, E_CHUNK), jnp.int32),
            pltpu.VMEM((KD, E_CHUNK), jnp.int32),
            pltpu.VMEM((KD, E_CHUNK, LANES), jnp.float32),
            pltpu.VMEM_SHARED((N_PAD, LANES), jnp.float32),
            pltpu.SemaphoreType.DMA,
        ],
        compiler_params=pltpu.CompilerParams(use_tc_tiling_on_sc=False),
    )


def _row_mask(i):
    rid = lax.broadcasted_iota(jnp.int32, (R_TC, 1), 0) + i * R_TC
    return rid < N_NODES


def _dot(a, b):
    return lax.dot_general(a, b, (((1,), (0,)), ((), ())),
                           preferred_element_type=jnp.float32)


def _k1_body(zd_ref, x_ref, y0_ref, dinv_ref):
    zd = zd_ref[...]
    deg = zd[0, :, 0:1] + zd[1, :, 0:1] + 1.0
    dinv = 1.0 / jnp.sqrt(deg)
    y0_ref[...] = x_ref[...] * dinv
    dinv_ref[...] = jnp.broadcast_to(dinv, dinv_ref.shape)


def _k2_body(z0_ref, y0_ref, dinv_ref, w1_ref, b1_ref, y1_ref):
    i = pl.program_id(0)
    z = z0_ref[...]
    dinv = dinv_ref[...]
    u = (z[0] + z[1] + y0_ref[...]) * dinv
    h = jnp.maximum(_dot(u, w1_ref[...]) + b1_ref[...], 0.0)
    y1 = h * dinv[:, 0:1]
    y1 = jnp.where(_row_mask(i), y1, 0.0)
    y1_ref[0] = y1[:, :LANES]
    y1_ref[1] = y1[:, LANES:]


def _k3_body(z1_ref, y1_ref, dinv_ref, w2_ref, b2_ref, w3_ref, y2_ref):
    i = pl.program_id(0)
    z = z1_ref[...]
    y1 = y1_ref[...]
    d1 = dinv_ref[...][:, 0:1]
    u = jnp.concatenate([z[0] + y1[0], z[1] + y1[1]], axis=1) * d1
    h2 = jnp.maximum(_dot(u, w2_ref[...]) + b2_ref[...], 0.0)
    t = _dot(h2, w3_ref[...])
    y2 = jnp.where(_row_mask(i), t * d1, 0.0)
    y2_ref[...] = y2


def _k4_body(z2_ref, y2_ref, dinv_ref, b3_ref, out_ref):
    z = z2_ref[...]
    d1 = dinv_ref[...][:, 0:1]
    v = (z[0] + z[1] + y2_ref[...]) * d1 + b3_ref[...]
    m = jnp.max(v, axis=1, keepdims=True)
    lse = jnp.log(jnp.sum(jnp.exp(v - m), axis=1, keepdims=True))
    out_ref[...] = v - m - lse


_GRID = (N_PAD // R_TC,)
_BS_N16 = pl.BlockSpec((R_TC, LANES), lambda i: (i, 0))
_BS_2N16 = pl.BlockSpec((2, R_TC, LANES), lambda i: (0, i, 0))


def _full(shape):
    return pl.BlockSpec(shape, lambda i: tuple(0 for _ in shape))


_deg_kernel = _make_scatter_kernel("deg")
_agg16_kernel = _make_scatter_kernel("agg16")
_agg32_kernel = _make_scatter_kernel("agg32")

_k1 = pl.pallas_call(
    _k1_body, grid=_GRID,
    in_specs=[_BS_2N16, _BS_N16],
    out_specs=[_BS_N16, _BS_N16],
    out_shape=[jax.ShapeDtypeStruct((N_PAD, LANES), jnp.float32)] * 2,
)
_k2 = pl.pallas_call(
    _k2_body, grid=_GRID,
    in_specs=[_BS_2N16, _BS_N16, _BS_N16, _full((LANES, 32)), _full((1, 32))],
    out_specs=_BS_2N16,
    out_shape=jax.ShapeDtypeStruct((2, N_PAD, LANES), jnp.float32),
)
_k3 = pl.pallas_call(
    _k3_body, grid=_GRID,
    in_specs=[_BS_2N16, _BS_2N16, _BS_N16, _full((32, 64)), _full((1, 64)),
              _full((64, LANES))],
    out_specs=_BS_N16,
    out_shape=jax.ShapeDtypeStruct((N_PAD, LANES), jnp.float32),
)
_k4 = pl.pallas_call(
    _k4_body, grid=_GRID,
    in_specs=[_BS_2N16, _BS_N16, _BS_N16, _full((1, LANES))],
    out_specs=_BS_N16,
    out_shape=jax.ShapeDtypeStruct((N_PAD, LANES), jnp.float32),
)


@jax.jit
def kernel(x, edge_index, W1, b1, W2, b2, W3, b3):
    e = edge_index.shape[1]
    src = edge_index[0].astype(jnp.int32)
    dst = edge_index[1].astype(jnp.int32)
    pad = jnp.full((E_ROWS * E_CHUNK - e,), DUMMY, jnp.int32)
    srcr = jnp.concatenate([src, pad]).reshape(E_ROWS, E_CHUNK)
    dstr = jnp.concatenate([dst, pad]).reshape(E_ROWS, E_CHUNK)

    x16 = jnp.pad(x, ((0, N_PAD - N_NODES), (0, LANES - x.shape[1])))
    e0 = jnp.concatenate(
        [jnp.ones((E_CHUNK, 1), jnp.float32),
         jnp.zeros((E_CHUNK, LANES - 1), jnp.float32)], axis=1)
    w1p = jnp.pad(W1, ((0, LANES - W1.shape[0]), (0, 0)))
    w3p = jnp.pad(W3, ((0, 0), (0, LANES - W3.shape[1])))
    b3p = jnp.concatenate([b3, jnp.full((LANES - b3.shape[0],), NEG, jnp.float32)])

    zdeg = _deg_kernel(x16[None], srcr, dstr, e0)
    y0, dinv = _k1(zdeg, x16)
    z0 = _agg16_kernel(y0[None], srcr, dstr, e0)
    y1h = _k2(z0, y0, dinv, w1p, b1.reshape(1, -1))
    z1 = _agg32_kernel(y1h, srcr, dstr, e0)
    y2 = _k3(z1, y1h, dinv, W2, b2.reshape(1, -1), w3p)
    z2 = _agg16_kernel(y2[None], srcr, dstr, e0)
    out = _k4(z2, y2, dinv, b3p.reshape(1, -1))
    return out[:N_NODES, :10]
